# baseline (speedup 1.0000x reference)
import functools

import jax
import jax.numpy as jnp
from jax import lax
from jax.experimental import pallas as pl
from jax.experimental.pallas import tpu as pltpu

N, D = 100000, 128
BN = 4096


def _scores_body(h_ref, w_ref, b_ref, out_ref):
    i = pl.program_id(0)
    hb = h_ref[...]
    wv = w_ref[...]
    z = lax.dot_general(wv, hb, (((0,), (1,)), ((), ())),
                        preferred_element_type=jnp.float32)
    z = z + b_ref[0, 0]
    s = jax.nn.sigmoid(z)
    col = i * BN + lax.broadcasted_iota(jnp.int32, (1, BN), 1)
    out_ref[...] = jnp.where(col < N, s, 0.0)


def _scores(h, W, b):
    grid = (N + BN - 1) // BN
    return pl.pallas_call(
        _scores_body,
        grid=(grid,),
        in_specs=[
            pl.BlockSpec((BN, D), lambda i: (i, 0)),
            pl.BlockSpec((D, 1), lambda i: (0, 0)),
            pl.BlockSpec((1, 1), lambda i: (0, 0)),
        ],
        out_specs=pl.BlockSpec((1, BN), lambda i: (0, i)),
        out_shape=jax.ShapeDtypeStruct((1, grid * BN), jnp.float32),
    )(h, W, b.reshape(1, 1))


def kernel(h, W, b, top_k):
    s = _scores(h, W, b)[0, :N]
    k = 50000
    top_vals, node_ids = jax.lax.top_k(s, k)
    new_h = h[node_ids] * top_vals[:, None]
    return (new_h, node_ids)


# speedup vs baseline: 1.2503x; 1.2503x over previous
import functools

import jax
import jax.numpy as jnp
from jax import lax
from jax.experimental import pallas as pl
from jax.experimental.pallas import tpu as pltpu
from jax.experimental.pallas import tpu_sc as plsc

N, D = 100000, 128
K = 50000
BN = 2048
NT = 16
NPAD = 100352
C = NPAD // NT
V = C // 16
RB = 6
R = 1 << RB
PASSES = 5
CROWS = C // 128
GW = 1568
GCH = 128
KPAD = 8 * C


def _scores_body(h_ref, w_ref, b_ref, out_ref):
    i = pl.program_id(0)
    z = lax.dot_general(w_ref[...], h_ref[...], (((0,), (1,)), ((), ())),
                        preferred_element_type=jnp.float32)
    s = jax.nn.sigmoid(z + b_ref[0, 0])
    col = i * BN + lax.broadcasted_iota(jnp.int32, (1, BN), 1)
    out_ref[...] = jnp.where(col < N, s, 0.0)


def _scores(h, W, b):
    grid = NPAD // BN
    return pl.pallas_call(
        _scores_body,
        grid=(grid,),
        in_specs=[
            pl.BlockSpec((BN, D), lambda i: (i, 0)),
            pl.BlockSpec((D, 1), lambda i: (0, 0)),
            pl.BlockSpec((1, 1), lambda i: (0, 0)),
        ],
        out_specs=pl.BlockSpec((1, BN), lambda i: (0, i)),
        out_shape=jax.ShapeDtypeStruct((1, NPAD), jnp.float32),
    )(h, W, b.reshape(1, 1))


_mesh = plsc.VectorSubcoreMesh(core_axis_name="c", subcore_axis_name="s")


@functools.partial(
    pl.kernel,
    out_type=(jax.ShapeDtypeStruct((K, D), jnp.float32),
              jax.ShapeDtypeStruct((KPAD,), jnp.int32)),
    mesh=_mesh,
    scratch_types=[
        pltpu.VMEM_SHARED((NPAD,), jnp.float32),
        pltpu.VMEM_SHARED((NPAD,), jnp.float32),
        pltpu.VMEM_SHARED((NPAD,), jnp.int32),
        pltpu.VMEM_SHARED((NPAD,), jnp.int32),
        pltpu.VMEM_SHARED((NT * R * 16,), jnp.int32),
        pltpu.VMEM((C,), jnp.float32),
        pltpu.VMEM((C,), jnp.int32),
        pltpu.VMEM((CROWS, 128), jnp.int32),
        pltpu.VMEM((CROWS, 128), jnp.float32),
        pltpu.VMEM((CROWS, 128), jnp.int32),
        pltpu.VMEM((NT * R * 16,), jnp.int32),
        pltpu.VMEM((R * 16,), jnp.int32),
        pltpu.VMEM((R * 16,), jnp.int32),
        pltpu.VMEM((R * 16,), jnp.int32),
        pltpu.SMEM((R,), jnp.int32),
        pltpu.VMEM((GCH,), jnp.int32),
        pltpu.VMEM((GCH,), jnp.float32),
        pltpu.VMEM((GCH, D), jnp.float32),
        pltpu.SemaphoreType.DMA,
        pltpu.SemaphoreType.DMA,
    ],
    compiler_params=pltpu.CompilerParams(needs_layout_passes=False),
)
def _sc_topk_gather(scores_hbm, h_hbm, newh_hbm, ids_hbm,
                    keyA, keyB, idA, idB, histS,
                    keyc, idc, posb, ksrc, isrc,
                    histall, histv, runb, bcomb, tots,
                    gidx, gval, grows, sem0, sem1):
    cid = lax.axis_index("c")
    sid = lax.axis_index("s")
    iota = lax.iota(jnp.int32, 16)
    iotaV = iota * V
    ones = jnp.ones((16,), jnp.int32)

    def zero1024(ref):
        def zb(i, carry):
            ref[pl.ds(i * 16, 16)] = jnp.zeros((16,), jnp.int32)
            return carry
        lax.fori_loop(0, R, zb, 0)

    def radix_pass(p, srcK, srcI, dstK, dstI):
        shift = RB * p
        pltpu.sync_copy(srcK.at[pl.ds(sid * C, C)], keyc)
        if p > 0:
            pltpu.sync_copy(srcI.at[pl.ds(sid * C, C)], idc)
        zero1024(histv)

        def pa(v, carry):
            kv = plsc.load_gather(keyc, [iotaV + v])
            kb = plsc.bitcast(kv, jnp.int32)
            d = (kb >> shift) & (R - 1)
            plsc.addupdate_scatter(histv, [d * 16 + iota], ones)
            return carry
        lax.fori_loop(0, V, pa, 0)

        pltpu.sync_copy(histv, histS.at[pl.ds(sid * R * 16, R * 16)])
        plsc.subcore_barrier()
        pltpu.sync_copy(histS, histall)

        def pb(d, carry):
            acc = jnp.int32(0)
            myexcl = jnp.zeros((16,), jnp.int32)
            for t in range(NT):
                hv = histall[pl.ds(t * R * 16 + d * 16, 16)]
                cs = plsc.cumsum(hv)
                excl = (cs - hv) + acc
                myexcl = jnp.where(sid == t, excl, myexcl)
                acc = acc + jnp.sum(hv)
            tots[d] = acc
            bcomb[pl.ds(d * 16, 16)] = myexcl
            return carry
        lax.fori_loop(0, R, pb, 0)

        def pbase(m, acc):
            d = (R - 1) - m
            t = tots[d]
            bcomb[pl.ds(d * 16, 16)] = bcomb[pl.ds(d * 16, 16)] + acc
            return acc + t
        lax.fori_loop(0, R, pbase, jnp.int32(0))

        zero1024(runb)

        def pc(r, carry):
            for q in range(8):
                v = r * 8 + q
                idxv = iotaV + v
                kv = plsc.load_gather(keyc, [idxv])
                kb = plsc.bitcast(kv, jnp.int32)
                if p == 0:
                    iv = sid * C + idxv
                else:
                    iv = plsc.load_gather(idc, [idxv])
                d = (kb >> shift) & (R - 1)
                slot = d * 16 + iota
                bv = plsc.load_gather(bcomb, [slot])
                rv = plsc.load_gather(runb, [slot])
                plsc.addupdate_scatter(runb, [slot], ones)
                posb[r, pl.ds(q * 16, 16)] = bv + rv
                ksrc[r, pl.ds(q * 16, 16)] = kv
                isrc[r, pl.ds(q * 16, 16)] = iv
            return carry
        lax.fori_loop(0, CROWS, pc, 0)

        for grp in range(7):
            cps = []
            for j7 in range(7):
                j = grp * 7 + j7
                cps.append(pltpu.async_copy(ksrc.at[j], dstK.at[posb.at[j]], sem0))
                cps.append(pltpu.async_copy(isrc.at[j], dstI.at[posb.at[j]], sem1))
            for cp in cps:
                cp.wait()
        plsc.subcore_barrier()

    radix_pass(0, scores_hbm, None, keyA, idA)
    radix_pass(1, keyA, idA, keyB, idB)
    radix_pass(2, keyB, idB, keyA, idA)
    radix_pass(3, keyA, idA, keyB, idB)
    radix_pass(4, keyB, idB, keyA, idA)

    @pl.when(jnp.logical_and(cid == 0, sid < 8))
    def _():
        pltpu.sync_copy(idA.at[pl.ds(sid * C, C)], ids_hbm.at[pl.ds(sid * C, C)])

    g = cid * NT + sid
    gbase = g * GW

    def scale_rows(nrows):
        def sr(r, carry):
            vv = plsc.load_gather(gval, [iota * 0 + r])
            for cc in range(8):
                grows[r, pl.ds(cc * 16, 16)] = grows[r, pl.ds(cc * 16, 16)] * vv
            return carry
        lax.fori_loop(0, nrows, sr, 0)

    for c in range(13):
        rb = gbase + c * GCH
        pltpu.sync_copy(idA.at[pl.ds(rb, GCH)], gidx)
        pltpu.sync_copy(keyA.at[pl.ds(rb, GCH)], gval)
        pltpu.async_copy(h_hbm.at[gidx], grows, sem0).wait()
        scale_rows(GCH)
        if c < 10:
            pltpu.sync_copy(grows, newh_hbm.at[pl.ds(rb, GCH)])
        elif c == 10:
            @pl.when(g < 31)
            def _():
                pltpu.sync_copy(grows, newh_hbm.at[pl.ds(rb, GCH)])

            @pl.when(g == 31)
            def _():
                pltpu.sync_copy(grows.at[pl.ds(0, 112)],
                                newh_hbm.at[pl.ds(rb, 112)])
        elif c == 11:
            @pl.when(g < 31)
            def _():
                pltpu.sync_copy(grows, newh_hbm.at[pl.ds(rb, GCH)])
        else:
            @pl.when(g < 31)
            def _():
                pltpu.sync_copy(grows.at[pl.ds(0, 32)],
                                newh_hbm.at[pl.ds(rb, 32)])


def kernel(h, W, b, top_k):
    scores_pad = _scores(h, W, b).reshape(NPAD)
    new_h, ids_pad = _sc_topk_gather(scores_pad, h)
    return (new_h, ids_pad[:K])


# speedup vs baseline: 1.4541x; 1.1630x over previous
import functools

import jax
import jax.numpy as jnp
from jax import lax
from jax.experimental import pallas as pl
from jax.experimental.pallas import tpu as pltpu
from jax.experimental.pallas import tpu_sc as plsc

N, D = 100000, 128
K = 50000
BN = 2048
NT = 16
NPAD = 100352
C = NPAD // NT
V = C // 16
CROWS = C // 128
RB = 8
R = 1 << RB
RV = R // 16
PASSES = 4
GW = 1568
GCH = 128
NCH = 13
KPAD = 8 * C


def _scores_body(h_ref, w_ref, b_ref, out_ref):
    i = pl.program_id(0)
    z = lax.dot_general(w_ref[...], h_ref[...], (((0,), (1,)), ((), ())),
                        preferred_element_type=jnp.float32)
    s = jax.nn.sigmoid(z + b_ref[0, 0])
    col = i * BN + lax.broadcasted_iota(jnp.int32, (1, BN), 1)
    out_ref[...] = jnp.where(col < N, s, 0.0)


def _scores(h, W, b):
    grid = NPAD // BN
    return pl.pallas_call(
        _scores_body,
        grid=(grid,),
        in_specs=[
            pl.BlockSpec((BN, D), lambda i: (i, 0)),
            pl.BlockSpec((D, 1), lambda i: (0, 0)),
            pl.BlockSpec((1, 1), lambda i: (0, 0)),
        ],
        out_specs=pl.BlockSpec((1, BN), lambda i: (0, i)),
        out_shape=jax.ShapeDtypeStruct((1, NPAD), jnp.float32),
    )(h, W, b.reshape(1, 1))


_mesh = plsc.VectorSubcoreMesh(core_axis_name="c", subcore_axis_name="s")


@functools.partial(
    pl.kernel,
    out_type=(jax.ShapeDtypeStruct((K, D), jnp.float32),
              jax.ShapeDtypeStruct((KPAD,), jnp.int32)),
    mesh=_mesh,
    scratch_types=[
        pltpu.VMEM_SHARED((NPAD,), jnp.int32),
        pltpu.VMEM_SHARED((NPAD,), jnp.int32),
        pltpu.VMEM_SHARED((NPAD,), jnp.int32),
        pltpu.VMEM_SHARED((NPAD,), jnp.int32),
        pltpu.VMEM_SHARED((NT * R,), jnp.int32),
        pltpu.VMEM((C,), jnp.float32),
        pltpu.VMEM((C,), jnp.int32),
        pltpu.VMEM((C,), jnp.int32),
        pltpu.VMEM((CROWS, 128), jnp.int32),
        pltpu.VMEM((CROWS, 128), jnp.int32),
        pltpu.VMEM((CROWS, 128), jnp.int32),
        pltpu.VMEM((16 * R,), jnp.int32),
        pltpu.VMEM((16 * R,), jnp.int32),
        pltpu.VMEM((NT * R,), jnp.int32),
        pltpu.VMEM((R,), jnp.int32),
        pltpu.VMEM((R,), jnp.int32),
        pltpu.VMEM((R,), jnp.int32),
        pltpu.VMEM((R,), jnp.int32),
        pltpu.VMEM((R,), jnp.int32),
        pltpu.VMEM((R,), jnp.int32),
        pltpu.VMEM((GCH,), jnp.int32),
        pltpu.VMEM((GCH,), jnp.int32),
        pltpu.VMEM((GCH,), jnp.int32),
        pltpu.VMEM((GCH,), jnp.int32),
        pltpu.VMEM((GCH, D), jnp.float32),
        pltpu.VMEM((GCH, D), jnp.float32),
        pltpu.SemaphoreType.DMA,
        pltpu.SemaphoreType.DMA,
    ],
    compiler_params=pltpu.CompilerParams(needs_layout_passes=False),
)
def _sc_topk_gather(scores_hbm, h_hbm, newh_hbm, ids_hbm,
                    keyA, keyB, idA, idB, tsumS,
                    keyc, keyi, idc, posb, ks, isrc,
                    histv, bcomb, tsall,
                    tsumv, totv, prev, inclv, basepre, accv,
                    gidx0, gidx1, gval0, gval1, grows0, grows1,
                    sem0, sem1):
    cid = lax.axis_index("c")
    sid = lax.axis_index("s")
    iota = lax.iota(jnp.int32, 16)
    iotaV = iota * V
    zer16 = jnp.zeros((16,), jnp.int32)
    one16 = jnp.ones((16,), jnp.int32)

    def zh(i, carry):
        histv[pl.ds(i * 16, 16)] = zer16
        return carry
    lax.fori_loop(0, R, zh, 0)

    def radix_pass(p, srcK, srcI, dstK, dstI):
        shift = RB * p

        if p == 0:
            pltpu.sync_copy(scores_hbm.at[pl.ds(sid * C, C)], keyc)
        else:
            pltpu.sync_copy(srcK.at[pl.ds(sid * C, C)], keyi)
            pltpu.sync_copy(srcI.at[pl.ds(sid * C, C)], idc)

        def load_key(idxv):
            if p == 0:
                return plsc.bitcast(plsc.load_gather(keyc, [idxv]), jnp.int32)
            return plsc.load_gather(keyi, [idxv])

        def pa(r, carry):
            for q in range(8):
                v = r * 8 + q
                kb = load_key(iotaV + v)
                d = (kb >> shift) & (R - 1)
                plsc.addupdate_scatter(histv, [iota * R + d], one16)
            return carry
        lax.fori_loop(0, CROWS, pa, 0)

        def ts(rv, carry):
            acc = zer16
            for l in range(16):
                acc = acc + histv[pl.ds(l * R + rv * 16, 16)]
            tsumv[pl.ds(rv * 16, 16)] = acc
            return carry
        lax.fori_loop(0, RV, ts, 0)
        pltpu.sync_copy(tsumv, tsumS.at[pl.ds(sid * R, R)])
        plsc.subcore_barrier()
        pltpu.sync_copy(tsumS, tsall)

        def pt(rv, carry):
            tot = zer16
            pre = zer16
            for t in range(16):
                hv = tsall[pl.ds(t * R + rv * 16, 16)]
                pre = pre + jnp.where(sid > t, hv, zer16)
                tot = tot + hv
            totv[pl.ds(rv * 16, 16)] = tot
            prev[pl.ds(rv * 16, 16)] = pre
            return carry
        lax.fori_loop(0, RV, pt, 0)

        def pi(rv, carry):
            tv = totv[pl.ds(rv * 16, 16)]
            cs = plsc.cumsum(tv)
            inclv[pl.ds(rv * 16, 16)] = cs + carry
            return carry + jnp.sum(tv)
        tall = lax.fori_loop(0, RV, pi, jnp.int32(0))

        def pb(rv, carry):
            basepre[pl.ds(rv * 16, 16)] = (
                tall - inclv[pl.ds(rv * 16, 16)] + prev[pl.ds(rv * 16, 16)])
            accv[pl.ds(rv * 16, 16)] = zer16
            return carry
        lax.fori_loop(0, RV, pb, 0)

        def pl_loop(l, carry):
            for rv in range(RV):
                hv = histv[pl.ds(l * R + rv * 16, 16)]
                av = accv[pl.ds(rv * 16, 16)]
                bcomb[pl.ds(l * R + rv * 16, 16)] = (
                    basepre[pl.ds(rv * 16, 16)] + av + hv)
                accv[pl.ds(rv * 16, 16)] = av + hv
            return carry
        lax.fori_loop(0, 16, pl_loop, 0)

        def pc(r, carry):
            for q in range(8):
                v = r * 8 + q
                idxv = iotaV + v
                kb = load_key(idxv)
                if p == 0:
                    iv = sid * C + idxv
                else:
                    iv = plsc.load_gather(idc, [idxv])
                d = (kb >> shift) & (R - 1)
                slot = iota * R + d
                bv = plsc.load_gather(bcomb, [slot])
                cnt = plsc.load_gather(histv, [slot])
                plsc.addupdate_scatter(histv, [slot], -one16)
                posb[r, pl.ds(q * 16, 16)] = bv - cnt
                ks[r, pl.ds(q * 16, 16)] = kb
                isrc[r, pl.ds(q * 16, 16)] = iv
            return carry
        lax.fori_loop(0, CROWS, pc, 0)

        for grp in range(7):
            cps = []
            for j7 in range(7):
                j = grp * 7 + j7
                cps.append(pltpu.async_copy(ks.at[j], dstK.at[posb.at[j]], sem0))
                cps.append(pltpu.async_copy(isrc.at[j], dstI.at[posb.at[j]], sem1))
            for cp in cps:
                cp.wait()
        plsc.subcore_barrier()

    radix_pass(0, None, None, keyA, idA)
    radix_pass(1, keyA, idA, keyB, idB)
    radix_pass(2, keyB, idB, keyA, idA)
    radix_pass(3, keyA, idA, keyB, idB)

    @pl.when(jnp.logical_and(cid == 0, sid < 8))
    def _():
        pltpu.sync_copy(idB.at[pl.ds(sid * C, C)], ids_hbm.at[pl.ds(sid * C, C)])

    g = cid * NT + sid
    gbase = g * GW

    def load_chunk(c, gidx, gval, grows, sem):
        rb = gbase + c * GCH
        pltpu.sync_copy(idB.at[pl.ds(rb, GCH)], gidx)
        pltpu.sync_copy(keyB.at[pl.ds(rb, GCH)], gval)
        return pltpu.async_copy(h_hbm.at[gidx], grows, sem)

    def scale_chunk(gval, grows):
        def sr(r, carry):
            vv = plsc.bitcast(
                plsc.load_gather(gval, [iota * 0 + r]), jnp.float32)
            for cc in range(8):
                grows[r, pl.ds(cc * 16, 16)] = grows[r, pl.ds(cc * 16, 16)] * vv
            return carry
        lax.fori_loop(0, GCH, sr, 0)

    def store_chunk(c, grows):
        rb = gbase + c * GCH
        if c < 10:
            pltpu.sync_copy(grows, newh_hbm.at[pl.ds(rb, GCH)])
        elif c == 10:
            @pl.when(g < 31)
            def _():
                pltpu.sync_copy(grows, newh_hbm.at[pl.ds(rb, GCH)])

            @pl.when(g == 31)
            def _():
                pltpu.sync_copy(grows.at[pl.ds(0, 112)],
                                newh_hbm.at[pl.ds(rb, 112)])
        else:
            @pl.when(g < 31)
            def _():
                pltpu.sync_copy(grows, newh_hbm.at[pl.ds(rb, GCH)])

    bufs = ((gidx0, gval0, grows0, sem0), (gidx1, gval1, grows1, sem1))
    cps = [None, None]
    cps[0] = load_chunk(0, *bufs[0])
    for c in range(NCH):
        nxt = c + 1
        if nxt < NCH:
            cps[nxt % 2] = load_chunk(nxt, *bufs[nxt % 2])
        cps[c % 2].wait()
        scale_chunk(bufs[c % 2][1], bufs[c % 2][2])
        store_chunk(c, bufs[c % 2][2])


def kernel(h, W, b, top_k):
    scores_pad = _scores(h, W, b).reshape(NPAD)
    new_h, ids_pad = _sc_topk_gather(scores_pad, h)
    return (new_h, ids_pad[:K])


# speedup vs baseline: 1.4559x; 1.0012x over previous
import functools

import jax
import jax.numpy as jnp
from jax import lax
from jax.experimental import pallas as pl
from jax.experimental.pallas import tpu as pltpu
from jax.experimental.pallas import tpu_sc as plsc

N, D = 100000, 128
K = 50000
BN = 2048
NT = 16
NPAD = 100352
C = NPAD // NT
V = C // 16
CROWS = C // 128
RB = 8
R = 1 << RB
RV = R // 16
PASSES = 4
GW = 1568
GCH = 128
NCH = 13
KPAD = 8 * C


def _scores_body(h_ref, w_ref, b_ref, out_ref):
    i = pl.program_id(0)
    z = lax.dot_general(w_ref[...], h_ref[...], (((0,), (1,)), ((), ())),
                        preferred_element_type=jnp.float32)
    s = jax.nn.sigmoid(z + b_ref[0, 0])
    col = i * BN + lax.broadcasted_iota(jnp.int32, (1, BN), 1)
    out_ref[...] = jnp.where(col < N, s, 0.0)


def _scores(h, W, b):
    grid = NPAD // BN
    return pl.pallas_call(
        _scores_body,
        grid=(grid,),
        in_specs=[
            pl.BlockSpec((BN, D), lambda i: (i, 0)),
            pl.BlockSpec((D, 1), lambda i: (0, 0)),
            pl.BlockSpec((1, 1), lambda i: (0, 0)),
        ],
        out_specs=pl.BlockSpec((1, BN), lambda i: (0, i)),
        out_shape=jax.ShapeDtypeStruct((1, NPAD), jnp.float32),
    )(h, W, b.reshape(1, 1))


_mesh = plsc.VectorSubcoreMesh(core_axis_name="c", subcore_axis_name="s")


@functools.partial(
    pl.kernel,
    out_type=(jax.ShapeDtypeStruct((K, D), jnp.float32),
              jax.ShapeDtypeStruct((KPAD,), jnp.int32)),
    mesh=_mesh,
    scratch_types=[
        pltpu.VMEM_SHARED((NPAD,), jnp.int32),
        pltpu.VMEM_SHARED((NPAD,), jnp.int32),
        pltpu.VMEM_SHARED((NPAD,), jnp.int32),
        pltpu.VMEM_SHARED((NPAD,), jnp.int32),
        pltpu.VMEM_SHARED((NT * R,), jnp.int32),
        pltpu.VMEM((C,), jnp.float32),
        pltpu.VMEM((C,), jnp.int32),
        pltpu.VMEM((C,), jnp.int32),
        pltpu.VMEM((CROWS, 128), jnp.int32),
        pltpu.VMEM((CROWS, 128), jnp.int32),
        pltpu.VMEM((CROWS, 128), jnp.int32),
        pltpu.VMEM((16 * R,), jnp.int32),
        pltpu.VMEM((16 * R,), jnp.int32),
        pltpu.VMEM((NT * R,), jnp.int32),
        pltpu.VMEM((R,), jnp.int32),
        pltpu.VMEM((R,), jnp.int32),
        pltpu.VMEM((R,), jnp.int32),
        pltpu.VMEM((R,), jnp.int32),
        pltpu.VMEM((R,), jnp.int32),
        pltpu.VMEM((R,), jnp.int32),
        pltpu.VMEM((GCH,), jnp.int32),
        pltpu.VMEM((GCH,), jnp.int32),
        pltpu.VMEM((GCH,), jnp.int32),
        pltpu.VMEM((GCH,), jnp.int32),
        pltpu.VMEM((GCH, D), jnp.float32),
        pltpu.VMEM((GCH, D), jnp.float32),
        pltpu.SemaphoreType.DMA,
        pltpu.SemaphoreType.DMA,
    ],
    compiler_params=pltpu.CompilerParams(needs_layout_passes=False),
)
def _sc_topk_gather(scores_hbm, h_hbm, newh_hbm, ids_hbm,
                    keyA, keyB, idA, idB, tsumS,
                    keyc, keyi, idc, posb, ks, isrc,
                    histv, bcomb, tsall,
                    tsumv, totv, prev, inclv, basepre, accv,
                    gidx0, gidx1, gval0, gval1, grows0, grows1,
                    sem0, sem1):
    cid = lax.axis_index("c")
    sid = lax.axis_index("s")
    iota = lax.iota(jnp.int32, 16)
    iotaV = iota * V
    zer16 = jnp.zeros((16,), jnp.int32)
    one16 = jnp.ones((16,), jnp.int32)

    def zh(i, carry):
        histv[pl.ds(i * 16, 16)] = zer16
        return carry
    lax.fori_loop(0, R, zh, 0)

    def radix_pass(p, srcK, srcI, dstK, dstI):
        shift = RB * p

        with jax.named_scope("ld%d" % p):
            if p == 0:
                pltpu.sync_copy(scores_hbm.at[pl.ds(sid * C, C)], keyc)
            else:
                pltpu.sync_copy(srcK.at[pl.ds(sid * C, C)], keyi)
                pltpu.sync_copy(srcI.at[pl.ds(sid * C, C)], idc)

        def load_key(idxv):
            if p == 0:
                return plsc.bitcast(plsc.load_gather(keyc, [idxv]), jnp.int32)
            return plsc.load_gather(keyi, [idxv])

        def pa(r, carry):
            for q in range(8):
                v = r * 8 + q
                kb = load_key(iotaV + v)
                d = (kb >> shift) & (R - 1)
                plsc.addupdate_scatter(histv, [iota * R + d], one16)
            return carry
        with jax.named_scope("hist%d" % p):
            lax.fori_loop(0, CROWS, pa, 0)

        def ts(rv, carry):
            acc = zer16
            for l in range(16):
                acc = acc + histv[pl.ds(l * R + rv * 16, 16)]
            tsumv[pl.ds(rv * 16, 16)] = acc
            return carry
        lax.fori_loop(0, RV, ts, 0)
        pltpu.sync_copy(tsumv, tsumS.at[pl.ds(sid * R, R)])
        plsc.subcore_barrier()
        pltpu.sync_copy(tsumS, tsall)

        def pt(rv, carry):
            tot = zer16
            pre = zer16
            for t in range(16):
                hv = tsall[pl.ds(t * R + rv * 16, 16)]
                pre = pre + jnp.where(sid > t, hv, zer16)
                tot = tot + hv
            totv[pl.ds(rv * 16, 16)] = tot
            prev[pl.ds(rv * 16, 16)] = pre
            return carry
        lax.fori_loop(0, RV, pt, 0)

        def pi(rv, carry):
            tv = totv[pl.ds(rv * 16, 16)]
            cs = plsc.cumsum(tv)
            inclv[pl.ds(rv * 16, 16)] = cs + carry
            return carry + jnp.sum(tv)
        tall = lax.fori_loop(0, RV, pi, jnp.int32(0))

        def pb(rv, carry):
            basepre[pl.ds(rv * 16, 16)] = (
                tall - inclv[pl.ds(rv * 16, 16)] + prev[pl.ds(rv * 16, 16)])
            accv[pl.ds(rv * 16, 16)] = zer16
            return carry
        lax.fori_loop(0, RV, pb, 0)

        def pl_loop(l, carry):
            for rv in range(RV):
                hv = histv[pl.ds(l * R + rv * 16, 16)]
                av = accv[pl.ds(rv * 16, 16)]
                bcomb[pl.ds(l * R + rv * 16, 16)] = (
                    basepre[pl.ds(rv * 16, 16)] + av + hv)
                accv[pl.ds(rv * 16, 16)] = av + hv
            return carry
        lax.fori_loop(0, 16, pl_loop, 0)

        def pc(r, carry):
            for q in range(8):
                v = r * 8 + q
                idxv = iotaV + v
                kb = load_key(idxv)
                if p == 0:
                    iv = sid * C + idxv
                else:
                    iv = plsc.load_gather(idc, [idxv])
                d = (kb >> shift) & (R - 1)
                slot = iota * R + d
                bv = plsc.load_gather(bcomb, [slot])
                cnt = plsc.load_gather(histv, [slot])
                plsc.addupdate_scatter(histv, [slot], -one16)
                posb[r, pl.ds(q * 16, 16)] = bv - cnt
                ks[r, pl.ds(q * 16, 16)] = kb
                isrc[r, pl.ds(q * 16, 16)] = iv
            return carry
        with jax.named_scope("rank%d" % p):
            lax.fori_loop(0, CROWS, pc, 0)

        with jax.named_scope("scat%d" % p):
         for grp in range(7):
            cps = []
            for j7 in range(7):
                j = grp * 7 + j7
                cps.append(pltpu.async_copy(ks.at[j], dstK.at[posb.at[j]], sem0))
                cps.append(pltpu.async_copy(isrc.at[j], dstI.at[posb.at[j]], sem1))
            for cp in cps:
                cp.wait()
        plsc.subcore_barrier()

    radix_pass(0, None, None, keyA, idA)
    radix_pass(1, keyA, idA, keyB, idB)
    radix_pass(2, keyB, idB, keyA, idA)
    radix_pass(3, keyA, idA, keyB, idB)

    @pl.when(jnp.logical_and(cid == 0, sid < 8))
    def _():
        pltpu.sync_copy(idB.at[pl.ds(sid * C, C)], ids_hbm.at[pl.ds(sid * C, C)])

    g = cid * NT + sid
    gbase = g * GW

    def load_chunk(c, gidx, gval, grows, sem):
        rb = gbase + c * GCH
        pltpu.sync_copy(idB.at[pl.ds(rb, GCH)], gidx)
        pltpu.sync_copy(keyB.at[pl.ds(rb, GCH)], gval)
        return pltpu.async_copy(h_hbm.at[gidx], grows, sem)

    def scale_chunk(gval, grows):
        def sr(r, carry):
            vv = plsc.bitcast(
                plsc.load_gather(gval, [iota * 0 + r]), jnp.float32)
            for cc in range(8):
                grows[r, pl.ds(cc * 16, 16)] = grows[r, pl.ds(cc * 16, 16)] * vv
            return carry
        lax.fori_loop(0, GCH, sr, 0)

    def store_chunk(c, grows):
        rb = gbase + c * GCH
        if c < 10:
            pltpu.sync_copy(grows, newh_hbm.at[pl.ds(rb, GCH)])
        elif c == 10:
            @pl.when(g < 31)
            def _():
                pltpu.sync_copy(grows, newh_hbm.at[pl.ds(rb, GCH)])

            @pl.when(g == 31)
            def _():
                pltpu.sync_copy(grows.at[pl.ds(0, 112)],
                                newh_hbm.at[pl.ds(rb, 112)])
        else:
            @pl.when(g < 31)
            def _():
                pltpu.sync_copy(grows, newh_hbm.at[pl.ds(rb, GCH)])

    bufs = ((gidx0, gval0, grows0, sem0), (gidx1, gval1, grows1, sem1))
    with jax.named_scope("gath"):
        cps = [None, None]
        cps[0] = load_chunk(0, *bufs[0])
        for c in range(NCH):
            nxt = c + 1
            if nxt < NCH:
                cps[nxt % 2] = load_chunk(nxt, *bufs[nxt % 2])
            cps[c % 2].wait()
            scale_chunk(bufs[c % 2][1], bufs[c % 2][2])
            store_chunk(c, bufs[c % 2][2])


def kernel(h, W, b, top_k):
    scores_pad = _scores(h, W, b).reshape(NPAD)
    new_h, ids_pad = _sc_topk_gather(scores_pad, h)
    return (new_h, ids_pad[:K])


# speedup vs baseline: 1.5003x; 1.0305x over previous
import functools

import jax
import jax.numpy as jnp
from jax import lax
from jax.experimental import pallas as pl
from jax.experimental.pallas import tpu as pltpu
from jax.experimental.pallas import tpu_sc as plsc

N, D = 100000, 128
K = 50000
BN = 2048
NT = 16
NPAD = 100352
C = NPAD // NT
V = C // 16
CROWS = C // 128
RB = 8
R = 1 << RB
RV = R // 16
PASSES = 4
GW = 1568
GCH = 128
NCH = 13
KPAD = 8 * C


def _scores_body(h_ref, w_ref, b_ref, out_ref):
    i = pl.program_id(0)
    z = lax.dot_general(w_ref[...], h_ref[...], (((0,), (1,)), ((), ())),
                        preferred_element_type=jnp.float32)
    s = jax.nn.sigmoid(z + b_ref[0, 0])
    col = i * BN + lax.broadcasted_iota(jnp.int32, (1, BN), 1)
    out_ref[...] = jnp.where(col < N, s, 0.0)


def _scores(h, W, b):
    grid = NPAD // BN
    return pl.pallas_call(
        _scores_body,
        grid=(grid,),
        in_specs=[
            pl.BlockSpec((BN, D), lambda i: (i, 0)),
            pl.BlockSpec((D, 1), lambda i: (0, 0)),
            pl.BlockSpec((1, 1), lambda i: (0, 0)),
        ],
        out_specs=pl.BlockSpec((1, BN), lambda i: (0, i)),
        out_shape=jax.ShapeDtypeStruct((1, NPAD), jnp.float32),
    )(h, W, b.reshape(1, 1))


_GDN = lax.GatherDimensionNumbers(
    offset_dims=(), collapsed_slice_dims=(0,), start_index_map=(0,))


def _take16(x, idx):
    return lax.gather(x, idx[:, None], _GDN, slice_sizes=(1,),
                      mode=lax.GatherScatterMode.PROMISE_IN_BOUNDS)


_mesh = plsc.VectorSubcoreMesh(core_axis_name="c", subcore_axis_name="s")


@functools.partial(
    pl.kernel,
    out_type=(jax.ShapeDtypeStruct((K, D), jnp.float32),
              jax.ShapeDtypeStruct((KPAD,), jnp.int32)),
    mesh=_mesh,
    scratch_types=[
        pltpu.VMEM_SHARED((NPAD,), jnp.int32),
        pltpu.VMEM_SHARED((NPAD,), jnp.int32),
        pltpu.VMEM_SHARED((NPAD,), jnp.int32),
        pltpu.VMEM_SHARED((NPAD,), jnp.int32),
        pltpu.VMEM_SHARED((NT * R,), jnp.int32),
        pltpu.VMEM((C,), jnp.float32),
        pltpu.VMEM((C,), jnp.int32),
        pltpu.VMEM((C,), jnp.int32),
        pltpu.VMEM((CROWS, 128), jnp.int32),
        pltpu.VMEM((CROWS, 128), jnp.int32),
        pltpu.VMEM((CROWS, 128), jnp.int32),
        pltpu.VMEM((16 * R,), jnp.int32),
        pltpu.VMEM((16 * R,), jnp.int32),
        pltpu.VMEM((NT * R,), jnp.int32),
        pltpu.VMEM((R,), jnp.int32),
        pltpu.VMEM((R,), jnp.int32),
        pltpu.VMEM((R,), jnp.int32),
        pltpu.VMEM((R,), jnp.int32),
        pltpu.VMEM((R,), jnp.int32),
        pltpu.VMEM((R,), jnp.int32),
        pltpu.VMEM((GCH,), jnp.int32),
        pltpu.VMEM((GCH,), jnp.int32),
        pltpu.VMEM((GCH,), jnp.int32),
        pltpu.VMEM((GCH,), jnp.int32),
        pltpu.VMEM((GCH, D), jnp.float32),
        pltpu.VMEM((GCH, D), jnp.float32),
        pltpu.SemaphoreType.DMA,
        pltpu.SemaphoreType.DMA,
        pltpu.SemaphoreType.DMA,
        pltpu.SemaphoreType.DMA,
    ],
    compiler_params=pltpu.CompilerParams(needs_layout_passes=False),
)
def _sc_topk_gather(scores_hbm, h_hbm, newh_hbm, ids_hbm,
                    keyA, keyB, idA, idB, tsumS,
                    keyc, keyi, idc, posb, ks, isrc,
                    histv, bcomb, tsall,
                    tsumv, totv, prev, inclv, basepre, accv,
                    gidx0, gidx1, gval0, gval1, grows0, grows1,
                    sem0, sem1, sem2, sem3):
    cid = lax.axis_index("c")
    sid = lax.axis_index("s")
    iota = lax.iota(jnp.int32, 16)
    iotaV = iota * V
    zer16 = jnp.zeros((16,), jnp.int32)
    one16 = jnp.ones((16,), jnp.int32)

    def zh(i, carry):
        histv[pl.ds(i * 16, 16)] = zer16
        return carry
    lax.fori_loop(0, R, zh, 0)

    def radix_pass(p, srcK, srcI, dstK, dstI):
        shift = RB * p

        with jax.named_scope("ld%d" % p):
            if p == 0:
                pltpu.sync_copy(scores_hbm.at[pl.ds(sid * C, C)], keyc)
            else:
                pltpu.sync_copy(srcK.at[pl.ds(sid * C, C)], keyi)
                pltpu.sync_copy(srcI.at[pl.ds(sid * C, C)], idc)

        def load_key(idxv):
            if p == 0:
                return plsc.bitcast(plsc.load_gather(keyc, [idxv]), jnp.int32)
            return plsc.load_gather(keyi, [idxv])

        def pa(r, carry):
            for q in range(8):
                v = r * 8 + q
                kb = load_key(iotaV + v)
                d = (kb >> shift) & (R - 1)
                plsc.addupdate_scatter(histv, [iota * R + (d ^ iota)], one16)
            return carry
        with jax.named_scope("hist%d" % p):
            lax.fori_loop(0, CROWS, pa, 0)

        def ts(rv, carry):
            acc = zer16
            for l in range(16):
                hv = histv[pl.ds(l * R + rv * 16, 16)]
                acc = acc + _take16(hv, iota ^ l)
            tsumv[pl.ds(rv * 16, 16)] = acc
            return carry
        lax.fori_loop(0, RV, ts, 0)
        pltpu.sync_copy(tsumv, tsumS.at[pl.ds(sid * R, R)])
        plsc.subcore_barrier()
        pltpu.sync_copy(tsumS, tsall)

        def pt(rv, carry):
            tot = zer16
            pre = zer16
            for t in range(16):
                hv = tsall[pl.ds(t * R + rv * 16, 16)]
                pre = pre + jnp.where(sid > t, hv, zer16)
                tot = tot + hv
            totv[pl.ds(rv * 16, 16)] = tot
            prev[pl.ds(rv * 16, 16)] = pre
            return carry
        lax.fori_loop(0, RV, pt, 0)

        def pi(rv, carry):
            tv = totv[pl.ds(rv * 16, 16)]
            cs = plsc.cumsum(tv)
            inclv[pl.ds(rv * 16, 16)] = cs + carry
            return carry + jnp.sum(tv)
        tall = lax.fori_loop(0, RV, pi, jnp.int32(0))

        def pb(rv, carry):
            basepre[pl.ds(rv * 16, 16)] = (
                tall - inclv[pl.ds(rv * 16, 16)] + prev[pl.ds(rv * 16, 16)])
            accv[pl.ds(rv * 16, 16)] = zer16
            return carry
        lax.fori_loop(0, RV, pb, 0)

        def pl_loop(l, carry):
            pi = iota ^ l
            for rv in range(RV):
                hv = _take16(histv[pl.ds(l * R + rv * 16, 16)], pi)
                av = accv[pl.ds(rv * 16, 16)]
                bcomb[pl.ds(l * R + rv * 16, 16)] = _take16(
                    basepre[pl.ds(rv * 16, 16)] + av + hv, pi)
                accv[pl.ds(rv * 16, 16)] = av + hv
            return carry
        lax.fori_loop(0, 16, pl_loop, 0)

        def pc(r, carry):
            for q in range(8):
                v = r * 8 + q
                idxv = iotaV + v
                kb = load_key(idxv)
                if p == 0:
                    iv = sid * C + idxv
                else:
                    iv = plsc.load_gather(idc, [idxv])
                d = (kb >> shift) & (R - 1)
                slot = iota * R + (d ^ iota)
                bv = plsc.load_gather(bcomb, [slot])
                cnt = plsc.load_gather(histv, [slot])
                plsc.addupdate_scatter(histv, [slot], -one16)
                posb[r, pl.ds(q * 16, 16)] = bv - cnt
                ks[r, pl.ds(q * 16, 16)] = kb
                isrc[r, pl.ds(q * 16, 16)] = iv
            return carry
        with jax.named_scope("rank%d" % p):
            lax.fori_loop(0, CROWS, pc, 0)

        with jax.named_scope("scat%d" % p):
         for grp in range(7):
            cps = []
            for j7 in range(7):
                j = grp * 7 + j7
                cps.append(pltpu.async_copy(ks.at[j], dstK.at[posb.at[j]], sem0))
                cps.append(pltpu.async_copy(isrc.at[j], dstI.at[posb.at[j]], sem1))
            for cp in cps:
                cp.wait()
        plsc.subcore_barrier()

    radix_pass(0, None, None, keyA, idA)
    radix_pass(1, keyA, idA, keyB, idB)
    radix_pass(2, keyB, idB, keyA, idA)
    radix_pass(3, keyA, idA, keyB, idB)

    @pl.when(jnp.logical_and(cid == 0, sid < 8))
    def _():
        pltpu.sync_copy(idB.at[pl.ds(sid * C, C)], ids_hbm.at[pl.ds(sid * C, C)])

    g = cid * NT + sid
    gbase = g * GW

    def load_chunk(c, gidx, gval, grows, sem):
        rb = gbase + c * GCH
        pltpu.sync_copy(idB.at[pl.ds(rb, GCH)], gidx)
        pltpu.sync_copy(keyB.at[pl.ds(rb, GCH)], gval)
        return pltpu.async_copy(h_hbm.at[gidx], grows, sem)

    def scale_chunk(gval, grows):
        def sr(r, carry):
            vv = plsc.bitcast(
                plsc.load_gather(gval, [iota * 0 + r]), jnp.float32)
            for cc in range(8):
                grows[r, pl.ds(cc * 16, 16)] = grows[r, pl.ds(cc * 16, 16)] * vv
            return carry
        lax.fori_loop(0, GCH, sr, 0)

    def store_chunk(c, grows, sem):
        rb = gbase + c * GCH
        if c < 10:
            return pltpu.async_copy(grows, newh_hbm.at[pl.ds(rb, GCH)], sem)
        if c == 10:
            @pl.when(g < 31)
            def _():
                pltpu.sync_copy(grows, newh_hbm.at[pl.ds(rb, GCH)])

            @pl.when(g == 31)
            def _():
                pltpu.sync_copy(grows.at[pl.ds(0, 112)],
                                newh_hbm.at[pl.ds(rb, 112)])
        else:
            @pl.when(g < 31)
            def _():
                pltpu.sync_copy(grows, newh_hbm.at[pl.ds(rb, GCH)])
        return None

    bufs = ((gidx0, gval0, grows0, sem0), (gidx1, gval1, grows1, sem1))
    stsem = (sem2, sem3)
    with jax.named_scope("gath"):
        cps = [None, None]
        stp = [None, None]
        cps[0] = load_chunk(0, *bufs[0])
        for c in range(NCH):
            nxt = c + 1
            if nxt < NCH:
                if stp[nxt % 2] is not None:
                    stp[nxt % 2].wait()
                    stp[nxt % 2] = None
                cps[nxt % 2] = load_chunk(nxt, *bufs[nxt % 2])
            cps[c % 2].wait()
            scale_chunk(bufs[c % 2][1], bufs[c % 2][2])
            stp[c % 2] = store_chunk(c, bufs[c % 2][2], stsem[c % 2])
        for b in range(2):
            if stp[b] is not None:
                stp[b].wait()


def kernel(h, W, b, top_k):
    scores_pad = _scores(h, W, b).reshape(NPAD)
    new_h, ids_pad = _sc_topk_gather(scores_pad, h)
    return (new_h, ids_pad[:K])


# speedup vs baseline: 1.5028x; 1.0017x over previous
import functools

import jax
import jax.numpy as jnp
from jax import lax
from jax.experimental import pallas as pl
from jax.experimental.pallas import tpu as pltpu
from jax.experimental.pallas import tpu_sc as plsc

N, D = 100000, 128
K = 50000
BN = 2048
NT = 16
NPAD = 100352
C = NPAD // NT
V = C // 16
CROWS = C // 128
RB = 8
R = 1 << RB
RV = R // 16
PASSES = 4
GW = 1568
GCH = 128
NCH = 13
KPAD = 8 * C


def _scores_body(h_ref, w_ref, b_ref, out_ref):
    i = pl.program_id(0)
    z = lax.dot_general(w_ref[...], h_ref[...], (((0,), (1,)), ((), ())),
                        preferred_element_type=jnp.float32)
    s = jax.nn.sigmoid(z + b_ref[0, 0])
    col = i * BN + lax.broadcasted_iota(jnp.int32, (1, BN), 1)
    out_ref[...] = jnp.where(col < N, s, 0.0)


def _scores(h, W, b):
    grid = NPAD // BN
    return pl.pallas_call(
        _scores_body,
        grid=(grid,),
        in_specs=[
            pl.BlockSpec((BN, D), lambda i: (i, 0)),
            pl.BlockSpec((D, 1), lambda i: (0, 0)),
            pl.BlockSpec((1, 1), lambda i: (0, 0)),
        ],
        out_specs=pl.BlockSpec((1, BN), lambda i: (0, i)),
        out_shape=jax.ShapeDtypeStruct((1, NPAD), jnp.float32),
    )(h, W, b.reshape(1, 1))


_GDN = lax.GatherDimensionNumbers(
    offset_dims=(), collapsed_slice_dims=(0,), start_index_map=(0,))


def _take16(x, idx):
    return lax.gather(x, idx[:, None], _GDN, slice_sizes=(1,),
                      mode=lax.GatherScatterMode.PROMISE_IN_BOUNDS)


_mesh = plsc.VectorSubcoreMesh(core_axis_name="c", subcore_axis_name="s")


@functools.partial(
    pl.kernel,
    out_type=(jax.ShapeDtypeStruct((K, D), jnp.float32),
              jax.ShapeDtypeStruct((KPAD,), jnp.int32)),
    mesh=_mesh,
    scratch_types=[
        pltpu.VMEM_SHARED((NPAD,), jnp.int32),
        pltpu.VMEM_SHARED((NPAD,), jnp.int32),
        pltpu.VMEM_SHARED((NPAD,), jnp.int32),
        pltpu.VMEM_SHARED((NPAD,), jnp.int32),
        pltpu.VMEM_SHARED((NT * R,), jnp.int32),
        pltpu.VMEM((C,), jnp.float32),
        pltpu.VMEM((C,), jnp.int32),
        pltpu.VMEM((C,), jnp.int32),
        pltpu.VMEM((CROWS, 128), jnp.int32),
        pltpu.VMEM((CROWS, 128), jnp.int32),
        pltpu.VMEM((CROWS, 128), jnp.int32),
        pltpu.VMEM((16 * R,), jnp.int32),
        pltpu.VMEM((16 * R,), jnp.int32),
        pltpu.VMEM((NT * R,), jnp.int32),
        pltpu.VMEM((R,), jnp.int32),
        pltpu.VMEM((R,), jnp.int32),
        pltpu.VMEM((R,), jnp.int32),
        pltpu.VMEM((R,), jnp.int32),
        pltpu.VMEM((R,), jnp.int32),
        pltpu.VMEM((R,), jnp.int32),
        pltpu.VMEM((GCH,), jnp.int32),
        pltpu.VMEM((GCH,), jnp.int32),
        pltpu.VMEM((GCH,), jnp.int32),
        pltpu.VMEM((GCH,), jnp.int32),
        pltpu.VMEM((GCH, D), jnp.float32),
        pltpu.VMEM((GCH, D), jnp.float32),
        pltpu.SemaphoreType.DMA,
        pltpu.SemaphoreType.DMA,
        pltpu.SemaphoreType.DMA,
        pltpu.SemaphoreType.DMA,
    ],
    compiler_params=pltpu.CompilerParams(needs_layout_passes=False),
)
def _sc_topk_gather(scores_hbm, h_hbm, newh_hbm, ids_hbm,
                    keyA, keyB, idA, idB, tsumS,
                    keyc, keyi, idc, posb, ks, isrc,
                    histv, bcomb, tsall,
                    tsumv, totv, prev, inclv, basepre, accv,
                    gidx0, gidx1, gval0, gval1, grows0, grows1,
                    sem0, sem1, sem2, sem3):
    cid = lax.axis_index("c")
    sid = lax.axis_index("s")
    iota = lax.iota(jnp.int32, 16)
    iotaV = iota * V
    zer16 = jnp.zeros((16,), jnp.int32)
    one16 = jnp.ones((16,), jnp.int32)

    def zh(i, carry):
        histv[pl.ds(i * 16, 16)] = zer16
        return carry
    lax.fori_loop(0, R, zh, 0)

    def radix_pass(p, srcK, srcI, dstK, dstI):
        shift = RB * p

        with jax.named_scope("ld%d" % p):
            if p == 0:
                pltpu.sync_copy(scores_hbm.at[pl.ds(sid * C, C)], keyc)
            else:
                pltpu.sync_copy(srcK.at[pl.ds(sid * C, C)], keyi)
                pltpu.sync_copy(srcI.at[pl.ds(sid * C, C)], idc)

        def load_key(idxv):
            if p == 0:
                return plsc.bitcast(plsc.load_gather(keyc, [idxv]), jnp.int32)
            return plsc.load_gather(keyi, [idxv])

        def pa(r, carry):
            for q in range(8):
                v = r * 8 + q
                kb = load_key(iotaV + v)
                d = (kb >> shift) & (R - 1)
                plsc.addupdate_scatter(histv, [iota * R + (d ^ iota)], one16)
            return carry
        with jax.named_scope("hist%d" % p):
            lax.fori_loop(0, CROWS, pa, 0)

        def ts(rv, carry):
            acc = zer16
            for l in range(16):
                hv = histv[pl.ds(l * R + rv * 16, 16)]
                acc = acc + _take16(hv, iota ^ l)
            tsumv[pl.ds(rv * 16, 16)] = acc
            return carry
        lax.fori_loop(0, RV, ts, 0)
        pltpu.sync_copy(tsumv, tsumS.at[pl.ds(sid * R, R)])
        plsc.subcore_barrier()
        pltpu.sync_copy(tsumS, tsall)

        def pt(rv, carry):
            tot = zer16
            pre = zer16
            for t in range(16):
                hv = tsall[pl.ds(t * R + rv * 16, 16)]
                pre = pre + jnp.where(sid > t, hv, zer16)
                tot = tot + hv
            totv[pl.ds(rv * 16, 16)] = tot
            prev[pl.ds(rv * 16, 16)] = pre
            return carry

        def pi_(rv, carry):
            tv = totv[pl.ds(rv * 16, 16)]
            cs = plsc.cumsum(tv)
            inclv[pl.ds(rv * 16, 16)] = cs + carry
            return carry + jnp.sum(tv)

        def pb(rv, carry):
            basepre[pl.ds(rv * 16, 16)] = (
                tall - inclv[pl.ds(rv * 16, 16)] + prev[pl.ds(rv * 16, 16)])
            accv[pl.ds(rv * 16, 16)] = zer16
            return carry

        def pl_loop(l, carry):
            pi = iota ^ l
            for rv in range(RV):
                hv = _take16(histv[pl.ds(l * R + rv * 16, 16)], pi)
                av = accv[pl.ds(rv * 16, 16)]
                bcomb[pl.ds(l * R + rv * 16, 16)] = _take16(
                    basepre[pl.ds(rv * 16, 16)] + av + hv, pi)
                accv[pl.ds(rv * 16, 16)] = av + hv
            return carry
        lax.fori_loop(0, RV, pt, 0)
        tall = lax.fori_loop(0, RV, pi_, jnp.int32(0))
        lax.fori_loop(0, RV, pb, 0)
        lax.fori_loop(0, 16, pl_loop, 0)

        def pc(r, carry):
            for q in range(8):
                v = r * 8 + q
                idxv = iotaV + v
                kb = load_key(idxv)
                if p == 0:
                    iv = sid * C + idxv
                else:
                    iv = plsc.load_gather(idc, [idxv])
                d = (kb >> shift) & (R - 1)
                slot = iota * R + (d ^ iota)
                bv = plsc.load_gather(bcomb, [slot])
                cnt = plsc.load_gather(histv, [slot])
                plsc.addupdate_scatter(histv, [slot], -one16)
                posb[r, pl.ds(q * 16, 16)] = bv - cnt
                ks[r, pl.ds(q * 16, 16)] = kb
                isrc[r, pl.ds(q * 16, 16)] = iv
            return carry
        with jax.named_scope("rank%d" % p):
            lax.fori_loop(0, CROWS, pc, 0)

        with jax.named_scope("scat%d" % p):
         for grp in range(7):
            cps = []
            for j7 in range(7):
                j = grp * 7 + j7
                cps.append(pltpu.async_copy(ks.at[j], dstK.at[posb.at[j]], sem0))
                cps.append(pltpu.async_copy(isrc.at[j], dstI.at[posb.at[j]], sem1))
            for cp in cps:
                cp.wait()
        plsc.subcore_barrier()

    radix_pass(0, None, None, keyA, idA)
    radix_pass(1, keyA, idA, keyB, idB)
    radix_pass(2, keyB, idB, keyA, idA)
    radix_pass(3, keyA, idA, keyB, idB)

    @pl.when(jnp.logical_and(cid == 0, sid < 8))
    def _():
        pltpu.sync_copy(idB.at[pl.ds(sid * C, C)], ids_hbm.at[pl.ds(sid * C, C)])

    g = cid * NT + sid
    gbase = g * GW

    def load_chunk(c, gidx, gval, grows, sem):
        rb = gbase + c * GCH
        pltpu.sync_copy(idB.at[pl.ds(rb, GCH)], gidx)
        pltpu.sync_copy(keyB.at[pl.ds(rb, GCH)], gval)
        return pltpu.async_copy(h_hbm.at[gidx], grows, sem)

    def scale_chunk(gval, grows):
        def sr(r, carry):
            vv = plsc.bitcast(
                plsc.load_gather(gval, [iota * 0 + r]), jnp.float32)
            for cc in range(8):
                grows[r, pl.ds(cc * 16, 16)] = grows[r, pl.ds(cc * 16, 16)] * vv
            return carry
        lax.fori_loop(0, GCH, sr, 0)

    def store_chunk(c, grows, sem):
        rb = gbase + c * GCH
        if c < 10:
            return pltpu.async_copy(grows, newh_hbm.at[pl.ds(rb, GCH)], sem)
        if c == 10:
            @pl.when(g < 31)
            def _():
                pltpu.sync_copy(grows, newh_hbm.at[pl.ds(rb, GCH)])

            @pl.when(g == 31)
            def _():
                pltpu.sync_copy(grows.at[pl.ds(0, 112)],
                                newh_hbm.at[pl.ds(rb, 112)])
        else:
            @pl.when(g < 31)
            def _():
                pltpu.sync_copy(grows, newh_hbm.at[pl.ds(rb, GCH)])
        return None

    bufs = ((gidx0, gval0, grows0, sem0), (gidx1, gval1, grows1, sem1))
    stsem = (sem2, sem3)
    with jax.named_scope("gath"):
        cps = [None, None]
        stp = [None, None]
        cps[0] = load_chunk(0, *bufs[0])
        for c in range(NCH):
            nxt = c + 1
            if nxt < NCH:
                if stp[nxt % 2] is not None:
                    stp[nxt % 2].wait()
                    stp[nxt % 2] = None
                cps[nxt % 2] = load_chunk(nxt, *bufs[nxt % 2])
            cps[c % 2].wait()
            scale_chunk(bufs[c % 2][1], bufs[c % 2][2])
            stp[c % 2] = store_chunk(c, bufs[c % 2][2], stsem[c % 2])
        for b in range(2):
            if stp[b] is not None:
                stp[b].wait()


def kernel(h, W, b, top_k):
    new_h, ids_pad = _sc_topk_gather(_scores(h, W, b).reshape(NPAD), h)
    return (new_h, ids_pad[:K])


# speedup vs baseline: 1.5040x; 1.0008x over previous
import functools

import jax
import jax.numpy as jnp
from jax import lax
from jax.experimental import pallas as pl
from jax.experimental.pallas import tpu as pltpu
from jax.experimental.pallas import tpu_sc as plsc

N, D = 100000, 128
K = 50000
BN = 2048
NT = 16
NPAD = 100352
C = NPAD // NT
V = C // 16
CROWS = C // 128
RB = 8
R = 1 << RB
RV = R // 16
PASSES = 4
GW = 1568
GCH = 128
NCH = 13
KPAD = 8 * C


def _scores_body(h_ref, w_ref, b_ref, out_ref):
    i = pl.program_id(0)
    z = lax.dot_general(w_ref[...], h_ref[...], (((0,), (1,)), ((), ())),
                        preferred_element_type=jnp.float32)
    z2 = z.reshape(BN // 128, 128)
    s = jax.nn.sigmoid(z2 + b_ref[0, 0])
    row = lax.broadcasted_iota(jnp.int32, (BN // 128, 128), 0)
    col = lax.broadcasted_iota(jnp.int32, (BN // 128, 128), 1)
    g = i * BN + row * 128 + col
    out_ref[...] = jnp.where(g < N, s, 0.0)


def _scores(h, W, b):
    grid = NPAD // BN
    return pl.pallas_call(
        _scores_body,
        grid=(grid,),
        in_specs=[
            pl.BlockSpec((BN, D), lambda i: (i, 0)),
            pl.BlockSpec((D, 1), lambda i: (0, 0)),
            pl.BlockSpec((1, 1), lambda i: (0, 0)),
        ],
        out_specs=pl.BlockSpec((BN // 128, 128), lambda i: (i, 0)),
        out_shape=jax.ShapeDtypeStruct((NPAD // 128, 128), jnp.float32),
    )(h, W, b.reshape(1, 1))


_GDN = lax.GatherDimensionNumbers(
    offset_dims=(), collapsed_slice_dims=(0,), start_index_map=(0,))


def _take16(x, idx):
    return lax.gather(x, idx[:, None], _GDN, slice_sizes=(1,),
                      mode=lax.GatherScatterMode.PROMISE_IN_BOUNDS)


_mesh = plsc.VectorSubcoreMesh(core_axis_name="c", subcore_axis_name="s")


@functools.partial(
    pl.kernel,
    out_type=(jax.ShapeDtypeStruct((K, D), jnp.float32),
              jax.ShapeDtypeStruct((KPAD,), jnp.int32)),
    mesh=_mesh,
    scratch_types=[
        pltpu.VMEM_SHARED((NPAD,), jnp.int32),
        pltpu.VMEM_SHARED((NPAD,), jnp.int32),
        pltpu.VMEM_SHARED((NPAD,), jnp.int32),
        pltpu.VMEM_SHARED((NPAD,), jnp.int32),
        pltpu.VMEM_SHARED((NT * R,), jnp.int32),
        pltpu.VMEM((C,), jnp.float32),
        pltpu.VMEM((C,), jnp.int32),
        pltpu.VMEM((C,), jnp.int32),
        pltpu.VMEM((CROWS, 128), jnp.int32),
        pltpu.VMEM((CROWS, 128), jnp.int32),
        pltpu.VMEM((CROWS, 128), jnp.int32),
        pltpu.VMEM((16 * R,), jnp.int32),
        pltpu.VMEM((16 * R,), jnp.int32),
        pltpu.VMEM((NT * R,), jnp.int32),
        pltpu.VMEM((R,), jnp.int32),
        pltpu.VMEM((R,), jnp.int32),
        pltpu.VMEM((R,), jnp.int32),
        pltpu.VMEM((R,), jnp.int32),
        pltpu.VMEM((R,), jnp.int32),
        pltpu.VMEM((R,), jnp.int32),
        pltpu.VMEM((GCH,), jnp.int32),
        pltpu.VMEM((GCH,), jnp.int32),
        pltpu.VMEM((GCH,), jnp.int32),
        pltpu.VMEM((GCH,), jnp.int32),
        pltpu.VMEM((GCH, D), jnp.float32),
        pltpu.VMEM((GCH, D), jnp.float32),
        pltpu.SemaphoreType.DMA,
        pltpu.SemaphoreType.DMA,
        pltpu.SemaphoreType.DMA,
        pltpu.SemaphoreType.DMA,
    ],
    compiler_params=pltpu.CompilerParams(needs_layout_passes=False),
)
def _sc_topk_gather(scores_hbm, h_hbm, newh_hbm, ids_hbm,
                    keyA, keyB, idA, idB, tsumS,
                    keyc, keyi, idc, posb, ks, isrc,
                    histv, bcomb, tsall,
                    tsumv, totv, prev, inclv, basepre, accv,
                    gidx0, gidx1, gval0, gval1, grows0, grows1,
                    sem0, sem1, sem2, sem3):
    cid = lax.axis_index("c")
    sid = lax.axis_index("s")
    iota = lax.iota(jnp.int32, 16)
    iotaV = iota * V
    zer16 = jnp.zeros((16,), jnp.int32)
    one16 = jnp.ones((16,), jnp.int32)

    def zh(i, carry):
        histv[pl.ds(i * 16, 16)] = zer16
        return carry
    lax.fori_loop(0, R, zh, 0)

    def radix_pass(p, srcK, srcI, dstK, dstI):
        shift = RB * p

        with jax.named_scope("ld%d" % p):
            if p == 0:
                pltpu.sync_copy(scores_hbm.at[pl.ds(sid * C, C)], keyc)
            else:
                pltpu.sync_copy(srcK.at[pl.ds(sid * C, C)], keyi)
                pltpu.sync_copy(srcI.at[pl.ds(sid * C, C)], idc)

        def load_key(idxv):
            if p == 0:
                return plsc.bitcast(plsc.load_gather(keyc, [idxv]), jnp.int32)
            return plsc.load_gather(keyi, [idxv])

        def pa(r, carry):
            for q in range(8):
                v = r * 8 + q
                kb = load_key(iotaV + v)
                d = (kb >> shift) & (R - 1)
                plsc.addupdate_scatter(histv, [iota * R + (d ^ iota)], one16)
            return carry
        with jax.named_scope("hist%d" % p):
            lax.fori_loop(0, CROWS, pa, 0)

        def ts(rv, carry):
            acc = zer16
            for l in range(16):
                hv = histv[pl.ds(l * R + rv * 16, 16)]
                acc = acc + _take16(hv, iota ^ l)
            tsumv[pl.ds(rv * 16, 16)] = acc
            return carry
        lax.fori_loop(0, RV, ts, 0)
        pltpu.sync_copy(tsumv, tsumS.at[pl.ds(sid * R, R)])
        plsc.subcore_barrier()
        pltpu.sync_copy(tsumS, tsall)

        def pt(rv, carry):
            tot = zer16
            pre = zer16
            for t in range(16):
                hv = tsall[pl.ds(t * R + rv * 16, 16)]
                pre = pre + jnp.where(sid > t, hv, zer16)
                tot = tot + hv
            totv[pl.ds(rv * 16, 16)] = tot
            prev[pl.ds(rv * 16, 16)] = pre
            return carry

        def pi_(rv, carry):
            tv = totv[pl.ds(rv * 16, 16)]
            cs = plsc.cumsum(tv)
            inclv[pl.ds(rv * 16, 16)] = cs + carry
            return carry + jnp.sum(tv)

        def pb(rv, carry):
            basepre[pl.ds(rv * 16, 16)] = (
                tall - inclv[pl.ds(rv * 16, 16)] + prev[pl.ds(rv * 16, 16)])
            accv[pl.ds(rv * 16, 16)] = zer16
            return carry

        def pl_loop(l, carry):
            pi = iota ^ l
            for rv in range(RV):
                hv = _take16(histv[pl.ds(l * R + rv * 16, 16)], pi)
                av = accv[pl.ds(rv * 16, 16)]
                bcomb[pl.ds(l * R + rv * 16, 16)] = _take16(
                    basepre[pl.ds(rv * 16, 16)] + av + hv, pi)
                accv[pl.ds(rv * 16, 16)] = av + hv
            return carry
        lax.fori_loop(0, RV, pt, 0)
        tall = lax.fori_loop(0, RV, pi_, jnp.int32(0))
        lax.fori_loop(0, RV, pb, 0)
        lax.fori_loop(0, 16, pl_loop, 0)

        def pc(r, carry):
            for q in range(8):
                v = r * 8 + q
                idxv = iotaV + v
                kb = load_key(idxv)
                if p == 0:
                    iv = sid * C + idxv
                else:
                    iv = plsc.load_gather(idc, [idxv])
                d = (kb >> shift) & (R - 1)
                slot = iota * R + (d ^ iota)
                bv = plsc.load_gather(bcomb, [slot])
                cnt = plsc.load_gather(histv, [slot])
                plsc.addupdate_scatter(histv, [slot], -one16)
                posb[r, pl.ds(q * 16, 16)] = bv - cnt
                ks[r, pl.ds(q * 16, 16)] = kb
                isrc[r, pl.ds(q * 16, 16)] = iv
            return carry
        with jax.named_scope("rank%d" % p):
            lax.fori_loop(0, CROWS, pc, 0)

        with jax.named_scope("scat%d" % p):
         for grp in range(7):
            cps = []
            for j7 in range(7):
                j = grp * 7 + j7
                cps.append(pltpu.async_copy(ks.at[j], dstK.at[posb.at[j]], sem0))
                cps.append(pltpu.async_copy(isrc.at[j], dstI.at[posb.at[j]], sem1))
            for cp in cps:
                cp.wait()
        plsc.subcore_barrier()

    radix_pass(0, None, None, keyA, idA)
    radix_pass(1, keyA, idA, keyB, idB)
    radix_pass(2, keyB, idB, keyA, idA)
    radix_pass(3, keyA, idA, keyB, idB)

    @pl.when(jnp.logical_and(cid == 0, sid < 8))
    def _():
        pltpu.sync_copy(idB.at[pl.ds(sid * C, C)], ids_hbm.at[pl.ds(sid * C, C)])

    g = cid * NT + sid
    gbase = g * GW

    def load_chunk(c, gidx, gval, grows, sem):
        rb = gbase + c * GCH
        pltpu.sync_copy(idB.at[pl.ds(rb, GCH)], gidx)
        pltpu.sync_copy(keyB.at[pl.ds(rb, GCH)], gval)
        return pltpu.async_copy(h_hbm.at[gidx], grows, sem)

    def scale_chunk(gval, grows):
        def sr(r, carry):
            vv = plsc.bitcast(
                plsc.load_gather(gval, [iota * 0 + r]), jnp.float32)
            for cc in range(8):
                grows[r, pl.ds(cc * 16, 16)] = grows[r, pl.ds(cc * 16, 16)] * vv
            return carry
        lax.fori_loop(0, GCH, sr, 0)

    def store_chunk(c, grows, sem):
        rb = gbase + c * GCH
        if c < 10:
            return pltpu.async_copy(grows, newh_hbm.at[pl.ds(rb, GCH)], sem)
        if c == 10:
            @pl.when(g < 31)
            def _():
                pltpu.sync_copy(grows, newh_hbm.at[pl.ds(rb, GCH)])

            @pl.when(g == 31)
            def _():
                pltpu.sync_copy(grows.at[pl.ds(0, 112)],
                                newh_hbm.at[pl.ds(rb, 112)])
        else:
            @pl.when(g < 31)
            def _():
                pltpu.sync_copy(grows, newh_hbm.at[pl.ds(rb, GCH)])
        return None

    bufs = ((gidx0, gval0, grows0, sem0), (gidx1, gval1, grows1, sem1))
    stsem = (sem2, sem3)
    with jax.named_scope("gath"):
        cps = [None, None]
        stp = [None, None]
        cps[0] = load_chunk(0, *bufs[0])
        for c in range(NCH):
            nxt = c + 1
            if nxt < NCH:
                if stp[nxt % 2] is not None:
                    stp[nxt % 2].wait()
                    stp[nxt % 2] = None
                cps[nxt % 2] = load_chunk(nxt, *bufs[nxt % 2])
            cps[c % 2].wait()
            scale_chunk(bufs[c % 2][1], bufs[c % 2][2])
            stp[c % 2] = store_chunk(c, bufs[c % 2][2], stsem[c % 2])
        for b in range(2):
            if stp[b] is not None:
                stp[b].wait()


def kernel(h, W, b, top_k):
    new_h, ids_pad = _sc_topk_gather(_scores(h, W, b).reshape(NPAD), h)
    return (new_h, ids_pad[:K])


# speedup vs baseline: 1.7015x; 1.1314x over previous
import functools

import jax
import jax.numpy as jnp
from jax import lax
from jax.experimental import pallas as pl
from jax.experimental.pallas import tpu as pltpu
from jax.experimental.pallas import tpu_sc as plsc

N, D = 100000, 128
K = 50000
BN = 7168
NT = 16
NPAD = 100352
C = NPAD // NT
V = C // 16
CROWS = C // 128
RB = 8
R = 1 << RB
RV = R // 16
PASSES = 4
GW = 1568
GCH = 128
NCH = 13
KPAD = 8 * C


def _scores_body(h_ref, w_ref, b_ref, out_ref):
    i = pl.program_id(0)
    z = lax.dot_general(w_ref[...], h_ref[...], (((0,), (1,)), ((), ())),
                        preferred_element_type=jnp.float32)
    z2 = z.reshape(BN // 128, 128)
    s = jax.nn.sigmoid(z2 + b_ref[0, 0])
    row = lax.broadcasted_iota(jnp.int32, (BN // 128, 128), 0)
    col = lax.broadcasted_iota(jnp.int32, (BN // 128, 128), 1)
    g = i * BN + row * 128 + col
    out_ref[...] = jnp.where(g < N, s, 0.0)


def _scores(h, W, b):
    grid = NPAD // BN
    return pl.pallas_call(
        _scores_body,
        grid=(grid,),
        in_specs=[
            pl.BlockSpec((BN, D), lambda i: (i, 0)),
            pl.BlockSpec((D, 1), lambda i: (0, 0)),
            pl.BlockSpec((1, 1), lambda i: (0, 0)),
        ],
        out_specs=pl.BlockSpec((BN // 128, 128), lambda i: (i, 0)),
        out_shape=jax.ShapeDtypeStruct((NPAD // 128, 128), jnp.float32),
    )(h, W, b.reshape(1, 1))


_GDN = lax.GatherDimensionNumbers(
    offset_dims=(), collapsed_slice_dims=(0,), start_index_map=(0,))


def _take16(x, idx):
    return lax.gather(x, idx[:, None], _GDN, slice_sizes=(1,),
                      mode=lax.GatherScatterMode.PROMISE_IN_BOUNDS)


_mesh = plsc.VectorSubcoreMesh(core_axis_name="c", subcore_axis_name="s")


@functools.partial(
    pl.kernel,
    out_type=(jax.ShapeDtypeStruct((K, D), jnp.float32),
              jax.ShapeDtypeStruct((KPAD,), jnp.int32)),
    mesh=_mesh,
    scratch_types=[
        pltpu.VMEM_SHARED((NPAD,), jnp.int32),
        pltpu.VMEM_SHARED((NPAD,), jnp.int32),
        pltpu.VMEM_SHARED((NPAD,), jnp.int32),
        pltpu.VMEM_SHARED((NPAD,), jnp.int32),
        pltpu.VMEM_SHARED((NT * R,), jnp.int32),
        pltpu.VMEM((C,), jnp.float32),
        pltpu.VMEM((C,), jnp.int32),
        pltpu.VMEM((C,), jnp.int32),
        pltpu.VMEM((CROWS, 128), jnp.int32),
        pltpu.VMEM((CROWS, 128), jnp.int32),
        pltpu.VMEM((CROWS, 128), jnp.int32),
        pltpu.VMEM((16 * R,), jnp.int32),
        pltpu.VMEM((16 * R,), jnp.int32),
        pltpu.VMEM((NT * R,), jnp.int32),
        pltpu.VMEM((R,), jnp.int32),
        pltpu.VMEM((R,), jnp.int32),
        pltpu.VMEM((R,), jnp.int32),
        pltpu.VMEM((R,), jnp.int32),
        pltpu.VMEM((R,), jnp.int32),
        pltpu.VMEM((R,), jnp.int32),
        pltpu.VMEM((GCH,), jnp.int32),
        pltpu.VMEM((GCH,), jnp.int32),
        pltpu.VMEM((GCH,), jnp.int32),
        pltpu.VMEM((GCH,), jnp.int32),
        pltpu.VMEM((GCH, D), jnp.float32),
        pltpu.VMEM((GCH, D), jnp.float32),
        pltpu.SemaphoreType.DMA,
        pltpu.SemaphoreType.DMA,
        pltpu.SemaphoreType.DMA,
        pltpu.SemaphoreType.DMA,
    ],
    compiler_params=pltpu.CompilerParams(needs_layout_passes=False),
)
def _sc_topk_gather(scores_hbm, h_hbm, newh_hbm, ids_hbm,
                    keyA, keyB, idA, idB, tsumS,
                    keyc, keyi, idc, posb, ks, isrc,
                    histv, bcomb, tsall,
                    tsumv, totv, prev, inclv, basepre, accv,
                    gidx0, gidx1, gval0, gval1, grows0, grows1,
                    sem0, sem1, sem2, sem3):
    cid = lax.axis_index("c")
    sid = lax.axis_index("s")
    iota = lax.iota(jnp.int32, 16)
    iotaV = iota * V
    zer16 = jnp.zeros((16,), jnp.int32)
    one16 = jnp.ones((16,), jnp.int32)

    def zh(i, carry):
        histv[pl.ds(i * 16, 16)] = zer16
        return carry
    lax.fori_loop(0, R, zh, 0)

    def radix_pass(p, srcK, srcI, dstK, dstI):
        shift = RB * p

        with jax.named_scope("ld%d" % p):
            if p == 0:
                pltpu.sync_copy(scores_hbm.at[pl.ds(sid * C, C)], keyc)
            else:
                pltpu.sync_copy(srcK.at[pl.ds(sid * C, C)], keyi)
                pltpu.sync_copy(srcI.at[pl.ds(sid * C, C)], idc)

        def load_key(idxv):
            if p == 0:
                return plsc.bitcast(plsc.load_gather(keyc, [idxv]), jnp.int32)
            return plsc.load_gather(keyi, [idxv])

        def pa(r, carry):
            for q in range(8):
                v = r * 8 + q
                kb = load_key(iotaV + v)
                d = (kb >> shift) & (R - 1)
                plsc.addupdate_scatter(histv, [iota * R + (d ^ iota)], one16)
            return carry
        with jax.named_scope("hist%d" % p):
            lax.fori_loop(0, CROWS, pa, 0)

        def ts(rv, carry):
            acc = zer16
            for l in range(16):
                hv = histv[pl.ds(l * R + rv * 16, 16)]
                acc = acc + _take16(hv, iota ^ l)
            tsumv[pl.ds(rv * 16, 16)] = acc
            return carry
        lax.fori_loop(0, RV, ts, 0)
        pltpu.sync_copy(tsumv, tsumS.at[pl.ds(sid * R, R)])
        plsc.subcore_barrier()
        pltpu.sync_copy(tsumS, tsall)

        def pt(rv, carry):
            tot = zer16
            pre = zer16
            for t in range(16):
                hv = tsall[pl.ds(t * R + rv * 16, 16)]
                pre = pre + jnp.where(sid > t, hv, zer16)
                tot = tot + hv
            totv[pl.ds(rv * 16, 16)] = tot
            prev[pl.ds(rv * 16, 16)] = pre
            return carry

        def pi_(rv, carry):
            tv = totv[pl.ds(rv * 16, 16)]
            cs = plsc.cumsum(tv)
            inclv[pl.ds(rv * 16, 16)] = cs + carry
            return carry + jnp.sum(tv)

        def pb(rv, carry):
            basepre[pl.ds(rv * 16, 16)] = (
                tall - inclv[pl.ds(rv * 16, 16)] + prev[pl.ds(rv * 16, 16)])
            accv[pl.ds(rv * 16, 16)] = zer16
            return carry

        def pl_loop(l, carry):
            pi = iota ^ l
            for rv in range(RV):
                hv = _take16(histv[pl.ds(l * R + rv * 16, 16)], pi)
                av = accv[pl.ds(rv * 16, 16)]
                bcomb[pl.ds(l * R + rv * 16, 16)] = _take16(
                    basepre[pl.ds(rv * 16, 16)] + av + hv, pi)
                accv[pl.ds(rv * 16, 16)] = av + hv
            return carry
        lax.fori_loop(0, RV, pt, 0)
        tall = lax.fori_loop(0, RV, pi_, jnp.int32(0))
        lax.fori_loop(0, RV, pb, 0)
        lax.fori_loop(0, 16, pl_loop, 0)

        def pc(r, carry):
            for q in range(8):
                v = r * 8 + q
                idxv = iotaV + v
                kb = load_key(idxv)
                if p == 0:
                    iv = sid * C + idxv
                else:
                    iv = plsc.load_gather(idc, [idxv])
                d = (kb >> shift) & (R - 1)
                slot = iota * R + (d ^ iota)
                bv = plsc.load_gather(bcomb, [slot])
                cnt = plsc.load_gather(histv, [slot])
                plsc.addupdate_scatter(histv, [slot], -one16)
                posb[r, pl.ds(q * 16, 16)] = bv - cnt
                ks[r, pl.ds(q * 16, 16)] = kb
                isrc[r, pl.ds(q * 16, 16)] = iv
            return carry
        with jax.named_scope("rank%d" % p):
            lax.fori_loop(0, CROWS, pc, 0)

        with jax.named_scope("scat%d" % p):
         for grp in range(1):
            cps = []
            for j in range(CROWS):
                cps.append(pltpu.async_copy(ks.at[j], dstK.at[posb.at[j]], sem0))
                cps.append(pltpu.async_copy(isrc.at[j], dstI.at[posb.at[j]], sem1))
            for cp in cps:
                cp.wait()
        plsc.subcore_barrier()

    radix_pass(0, None, None, keyA, idA)
    radix_pass(1, keyA, idA, keyB, idB)
    radix_pass(2, keyB, idB, keyA, idA)
    radix_pass(3, keyA, idA, keyB, idB)

    @pl.when(jnp.logical_and(cid == 0, sid < 8))
    def _():
        pltpu.sync_copy(idB.at[pl.ds(sid * C, C)], ids_hbm.at[pl.ds(sid * C, C)])

    g = cid * NT + sid
    gbase = g * GW

    def load_chunk(c, gidx, gval, grows, sem):
        rb = gbase + c * GCH
        pltpu.sync_copy(idB.at[pl.ds(rb, GCH)], gidx)
        pltpu.sync_copy(keyB.at[pl.ds(rb, GCH)], gval)
        return pltpu.async_copy(h_hbm.at[gidx], grows, sem)

    def scale_chunk(gval, grows):
        def sr(r, carry):
            vv = plsc.bitcast(
                plsc.load_gather(gval, [iota * 0 + r]), jnp.float32)
            for cc in range(8):
                grows[r, pl.ds(cc * 16, 16)] = grows[r, pl.ds(cc * 16, 16)] * vv
            return carry
        lax.fori_loop(0, GCH, sr, 0)

    def store_chunk(c, grows, sem):
        rb = gbase + c * GCH
        if c < 10:
            return pltpu.async_copy(grows, newh_hbm.at[pl.ds(rb, GCH)], sem)
        if c == 10:
            @pl.when(g < 31)
            def _():
                pltpu.sync_copy(grows, newh_hbm.at[pl.ds(rb, GCH)])

            @pl.when(g == 31)
            def _():
                pltpu.sync_copy(grows.at[pl.ds(0, 112)],
                                newh_hbm.at[pl.ds(rb, 112)])
        else:
            @pl.when(g < 31)
            def _():
                pltpu.sync_copy(grows, newh_hbm.at[pl.ds(rb, GCH)])
        return None

    bufs = ((gidx0, gval0, grows0, sem0), (gidx1, gval1, grows1, sem1))
    stsem = (sem2, sem3)
    with jax.named_scope("gath"):
        cps = [None, None]
        stp = [None, None]
        cps[0] = load_chunk(0, *bufs[0])
        for c in range(NCH):
            nxt = c + 1
            if nxt < NCH:
                if stp[nxt % 2] is not None:
                    stp[nxt % 2].wait()
                    stp[nxt % 2] = None
                cps[nxt % 2] = load_chunk(nxt, *bufs[nxt % 2])
            cps[c % 2].wait()
            scale_chunk(bufs[c % 2][1], bufs[c % 2][2])
            stp[c % 2] = store_chunk(c, bufs[c % 2][2], stsem[c % 2])
        for b in range(2):
            if stp[b] is not None:
                stp[b].wait()


def kernel(h, W, b, top_k):
    new_h, ids_pad = _sc_topk_gather(_scores(h, W, b).reshape(NPAD), h)
    return (new_h, ids_pad[:K])
